# baseline (speedup 1.0000x reference)
import functools

import jax
import jax.numpy as jnp
from jax import lax
from jax.experimental import pallas as pl
from jax.experimental.pallas import tpu as pltpu
from jax.experimental.pallas import tpu_sc as plsc

B = 4096
D = 2048
H = 8192
K = 256


_ENC_RBLK = 512
_ENC_HBLK = 1024


def _enc_body(x_ref, w_ref, b_ref, out_ref):
    acc = lax.dot_general(
        x_ref[...], w_ref[...], (((1,), (1,)), ((), ())),
        preferred_element_type=jnp.float32,
    )
    out_ref[...] = acc + b_ref[...]


def _encoder(x, W1, b1_2d):
    grid = (B // _ENC_RBLK, H // _ENC_HBLK)
    return pl.pallas_call(
        _enc_body,
        grid=grid,
        in_specs=[
            pl.BlockSpec((_ENC_RBLK, D), lambda r, h: (r, 0)),
            pl.BlockSpec((_ENC_HBLK, D), lambda r, h: (h, 0)),
            pl.BlockSpec((1, _ENC_HBLK), lambda r, h: (0, h)),
        ],
        out_specs=pl.BlockSpec((_ENC_RBLK, _ENC_HBLK), lambda r, h: (r, h)),
        out_shape=jax.ShapeDtypeStruct((B, H), jnp.float32),
        compiler_params=pltpu.CompilerParams(
            dimension_semantics=("parallel", "arbitrary")),
    )(x, W1, b1_2d)


_THR_RBLK = 256


def _thr_body(enc_ref, thr_ref, mu_ref):
    y = lax.bitcast_convert_type(enc_ref[...], jnp.uint32)
    neg = y >= jnp.uint32(0x80000000)
    mu = jnp.where(neg, ~y, y | jnp.uint32(0x80000000))
    mu_ref[...] = mu

    lo0 = jnp.zeros((_THR_RBLK, 1), jnp.uint32)
    hi0 = jnp.full((_THR_RBLK, 1), jnp.uint32(0xFFFFFFFF))

    def step(_, carry):
        lo, hi = carry
        mid = lo + ((hi - lo) // jnp.uint32(2)) + ((hi - lo) % jnp.uint32(2))
        cnt = jnp.sum((mu_ref[...] >= mid).astype(jnp.int32), axis=1,
                      keepdims=True)
        ge = cnt >= K
        lo = jnp.where(ge, mid, lo)
        hi = jnp.where(ge, hi, mid - jnp.uint32(1))
        return lo, hi

    lo, hi = lax.fori_loop(0, 32, step, (lo0, hi0))
    code = lo
    pos = code >= jnp.uint32(0x80000000)
    y_out = jnp.where(pos, code ^ jnp.uint32(0x80000000), ~code)
    thr_ref[...] = lax.bitcast_convert_type(y_out, jnp.float32)


def _thresholds(enc):
    return pl.pallas_call(
        _thr_body,
        grid=(B // _THR_RBLK,),
        in_specs=[pl.BlockSpec((_THR_RBLK, H), lambda r: (r, 0))],
        out_specs=pl.BlockSpec((_THR_RBLK, 1), lambda r: (r, 0)),
        out_shape=jax.ShapeDtypeStruct((B, 1), jnp.float32),
        scratch_shapes=[pltpu.VMEM((_THR_RBLK, H), jnp.uint32)],
        compiler_params=pltpu.CompilerParams(
            dimension_semantics=("arbitrary",)),
    )(enc)


_NW = 32
_RPW = B // _NW


def _sc_thr_body(enc_hbm, thr_hbm, rowa, rowb, hist, stage, sema, semb):
    cid = lax.axis_index("c")
    sid = lax.axis_index("s")
    wid = sid * 2 + cid
    base_row = wid * _RPW

    lanes = lax.iota(jnp.int32, 16)
    ONES = jnp.ones((16,), jnp.int32)
    ZEROS = jnp.zeros((16,), jnp.int32)
    NEGINF = jnp.full((16,), jnp.int32(-2147483648))

    def monotone(y):
        return y ^ lax.shift_right_logical(
            lax.shift_right_arithmetic(y, 31), 1)

    def zero_hist(nblk):
        def z(i, c):
            hist[pl.ds(i * 16, 16)] = ZEROS
            return c
        lax.fori_loop(0, nblk, z, 0)

    def pass_hist(row_ref, key_fn, sel_fn):
        def body(i, mx):
            m = monotone(row_ref[pl.ds(i * 16, 16)])
            kv = key_fn(m)
            if sel_fn is None:
                plsc.addupdate_scatter(hist, [kv], ONES)
                return jnp.maximum(mx, kv)
            s = sel_fn(m)
            plsc.addupdate_scatter(hist, [kv], ONES, mask=s)
            return jnp.where(s, jnp.maximum(mx, kv), mx)
        mx = lax.fori_loop(0, H // 16, body, NEGINF)
        return jnp.max(mx)

    def find_bucket(jstart, Kr):
        def cond(c):
            return jnp.logical_not(c[2])

        def bodyw(c):
            j, T, _ = c
            h = hist[pl.ds(j * 16, 16)]
            bs = jnp.sum(h)
            hit = (T + bs) >= Kr
            return (jnp.where(hit, j, j - 1), jnp.where(hit, T, T + bs), hit)

        j, T, _ = lax.while_loop(cond, bodyw,
                                 (jstart, jnp.int32(0), False))
        h = hist[pl.ds(j * 16, 16)]
        P = plsc.cumsum(h)
        total = jnp.max(P)
        cnt_ge = (T + total) - P + h
        maskv = cnt_ge >= Kr
        rev = lax.rev(maskv.astype(jnp.int32), (0,)) != 0
        l_vec = 15 - plsc.all_reduce_ffs(rev)
        Psel = jnp.max(jnp.where(lanes == l_vec, P, 0))
        cnt_gt = (T + total) - Psel
        return j * 16 + jnp.max(l_vec), Kr - cnt_gt

    def process_row(row_ref, r_local):
        zero_hist(256)
        key1 = lambda m: lax.shift_right_arithmetic(m, 20) + 2048
        mx1 = pass_hist(row_ref, key1, None)
        b1, K1 = find_bucket(lax.shift_right_arithmetic(mx1, 4), K)

        zero_hist(256)
        key2 = lambda m: lax.shift_right_logical(m, 8) & 0xFFF
        mx2 = pass_hist(row_ref, key2, lambda m: key1(m) == b1)
        b2, K2 = find_bucket(lax.shift_right_arithmetic(mx2, 4), K1)

        zero_hist(16)
        key3 = lambda m: m & 0xFF
        mx3 = pass_hist(row_ref, key3,
                        lambda m: (key1(m) == b1) & (key2(m) == b2))
        b3, _ = find_bucket(lax.shift_right_arithmetic(mx3, 4), K2)

        m_thr = ((b1 - 2048) << 20) | (b2 << 8) | b3
        mv = m_thr + ZEROS
        yv = jnp.where(mv >= 0, mv, mv ^ 0x7FFFFFFF)
        plsc.store_scatter(stage, [r_local + ZEROS], yv, mask=lanes == 0)

    pltpu.make_async_copy(enc_hbm.at[base_row], rowa, sema).start()
    pltpu.make_async_copy(enc_hbm.at[base_row + 1], rowb, semb).start()

    def outer(i, c):
        for half, (buf, sem) in enumerate(((rowa, sema), (rowb, semb))):
            r = i * 2 + half
            pltpu.make_async_copy(enc_hbm.at[base_row + r], buf, sem).wait()
            process_row(buf, r)

            @pl.when(r + 2 < _RPW)
            def _():
                pltpu.make_async_copy(
                    enc_hbm.at[base_row + r + 2], buf, sem).start()
        return c

    lax.fori_loop(0, _RPW // 2, outer, 0)
    pltpu.sync_copy(stage, thr_hbm.at[pl.ds(base_row, _RPW)])


def _thresholds_sc(enc):
    mesh = plsc.VectorSubcoreMesh(core_axis_name="c", subcore_axis_name="s")
    k = functools.partial(
        pl.kernel,
        mesh=mesh,
        compiler_params=pltpu.CompilerParams(needs_layout_passes=False),
        out_type=jax.ShapeDtypeStruct((B,), jnp.int32),
        scratch_types=[
            pltpu.VMEM((H,), jnp.int32),
            pltpu.VMEM((H,), jnp.int32),
            pltpu.VMEM((4096,), jnp.int32),
            pltpu.VMEM((_RPW,), jnp.int32),
            pltpu.SemaphoreType.DMA,
            pltpu.SemaphoreType.DMA,
        ],
    )(_sc_thr_body)
    enc_i32 = lax.bitcast_convert_type(enc, jnp.int32)
    return lax.bitcast_convert_type(k(enc_i32), jnp.float32)


_DEC_RBLK = 512
_DEC_HBLK = 512
_DEC_HSTEPS = H // _DEC_HBLK


def _dec_body(enc_ref, thr_ref, w2_ref, b2_ref, sp_ref, dec_ref):
    h = pl.program_id(1)
    sp = jnp.where(enc_ref[...] >= thr_ref[...], enc_ref[...], 0.0)
    sp_ref[...] = sp
    part = lax.dot_general(
        sp, w2_ref[...], (((1,), (1,)), ((), ())),
        preferred_element_type=jnp.float32,
    )

    @pl.when(h == 0)
    def _():
        dec_ref[...] = part

    @pl.when(h != 0)
    def _():
        dec_ref[...] += part

    @pl.when(h == _DEC_HSTEPS - 1)
    def _():
        dec_ref[...] = jax.nn.sigmoid(dec_ref[...] + b2_ref[...])


def _decoder(enc, thr, W2, b2_2d):
    grid = (B // _DEC_RBLK, _DEC_HSTEPS)
    return pl.pallas_call(
        _dec_body,
        grid=grid,
        in_specs=[
            pl.BlockSpec((_DEC_RBLK, _DEC_HBLK), lambda r, h: (r, h)),
            pl.BlockSpec((_DEC_RBLK, 1), lambda r, h: (r, 0)),
            pl.BlockSpec((D, _DEC_HBLK), lambda r, h: (0, h)),
            pl.BlockSpec((1, D), lambda r, h: (0, 0)),
        ],
        out_specs=[
            pl.BlockSpec((_DEC_RBLK, _DEC_HBLK), lambda r, h: (r, h)),
            pl.BlockSpec((_DEC_RBLK, D), lambda r, h: (r, 0)),
        ],
        out_shape=[
            jax.ShapeDtypeStruct((B, H), jnp.float32),
            jax.ShapeDtypeStruct((B, D), jnp.float32),
        ],
        compiler_params=pltpu.CompilerParams(
            dimension_semantics=("parallel", "arbitrary")),
    )(enc, thr, W2, b2_2d)


def kernel(x, W1, b1, W2, b2):
    x = x.reshape(B, D)
    enc = _encoder(x, W1, b1.reshape(1, H))
    thr = _thresholds_sc(enc).reshape(B, 1)
    sparse, decoded = _decoder(enc, thr, W2, b2.reshape(1, D))
    return decoded, sparse


# speedup vs baseline: 2.3644x; 2.3644x over previous
import functools

import jax
import jax.numpy as jnp
from jax import lax
from jax.experimental import pallas as pl
from jax.experimental.pallas import tpu as pltpu
from jax.experimental.pallas import tpu_sc as plsc

B = 4096
D = 2048
H = 8192
K = 256


_ENC_RBLK = 512
_ENC_HBLK = 1024


def _enc_body(x_ref, w_ref, b_ref, out_ref):
    acc = lax.dot_general(
        x_ref[...], w_ref[...], (((1,), (1,)), ((), ())),
        preferred_element_type=jnp.float32,
    )
    out_ref[...] = acc + b_ref[...]


def _encoder(x, W1, b1_2d):
    grid = (B // _ENC_RBLK, H // _ENC_HBLK)
    return pl.pallas_call(
        _enc_body,
        grid=grid,
        in_specs=[
            pl.BlockSpec((_ENC_RBLK, D), lambda r, h: (r, 0)),
            pl.BlockSpec((_ENC_HBLK, D), lambda r, h: (h, 0)),
            pl.BlockSpec((1, _ENC_HBLK), lambda r, h: (0, h)),
        ],
        out_specs=pl.BlockSpec((_ENC_RBLK, _ENC_HBLK), lambda r, h: (r, h)),
        out_shape=jax.ShapeDtypeStruct((B, H), jnp.float32),
        compiler_params=pltpu.CompilerParams(
            dimension_semantics=("parallel", "arbitrary")),
    )(x, W1, b1_2d)


_THR_RBLK = 256


def _thr_body(enc_ref, thr_ref, mu_ref):
    y = lax.bitcast_convert_type(enc_ref[...], jnp.uint32)
    neg = y >= jnp.uint32(0x80000000)
    mu = jnp.where(neg, ~y, y | jnp.uint32(0x80000000))
    mu_ref[...] = mu

    lo0 = jnp.zeros((_THR_RBLK, 1), jnp.uint32)
    hi0 = jnp.full((_THR_RBLK, 1), jnp.uint32(0xFFFFFFFF))

    def step(_, carry):
        lo, hi = carry
        mid = lo + ((hi - lo) // jnp.uint32(2)) + ((hi - lo) % jnp.uint32(2))
        cnt = jnp.sum((mu_ref[...] >= mid).astype(jnp.int32), axis=1,
                      keepdims=True)
        ge = cnt >= K
        lo = jnp.where(ge, mid, lo)
        hi = jnp.where(ge, hi, mid - jnp.uint32(1))
        return lo, hi

    lo, hi = lax.fori_loop(0, 32, step, (lo0, hi0))
    code = lo
    pos = code >= jnp.uint32(0x80000000)
    y_out = jnp.where(pos, code ^ jnp.uint32(0x80000000), ~code)
    thr_ref[...] = lax.bitcast_convert_type(y_out, jnp.float32)


def _thresholds(enc):
    return pl.pallas_call(
        _thr_body,
        grid=(B // _THR_RBLK,),
        in_specs=[pl.BlockSpec((_THR_RBLK, H), lambda r: (r, 0))],
        out_specs=pl.BlockSpec((_THR_RBLK, 1), lambda r: (r, 0)),
        out_shape=jax.ShapeDtypeStruct((B, 1), jnp.float32),
        scratch_shapes=[pltpu.VMEM((_THR_RBLK, H), jnp.uint32)],
        compiler_params=pltpu.CompilerParams(
            dimension_semantics=("arbitrary",)),
    )(enc)


_NW = 32
_RPW = B // _NW


_UNROLL = 8


def _sc_thr_body(enc_hbm, thr_hbm, rowa, rowb, hist, val, stage, sema, semb):
    cid = lax.axis_index("c")
    sid = lax.axis_index("s")
    wid = sid * 2 + cid
    base_row = wid * _RPW

    lanes = lax.iota(jnp.int32, 16)
    ONES = jnp.ones((16,), jnp.int32)
    ZEROS = jnp.zeros((16,), jnp.int32)
    NEGINF = jnp.full((16,), jnp.int32(-2147483648))

    def monotone(y):
        return y ^ lax.shift_right_logical(
            lax.shift_right_arithmetic(y, 31), 1)

    def zero_hist(nblk):
        def z(i, c):
            for u in range(_UNROLL):
                hist[pl.ds((i * _UNROLL + u) * 16, 16)] = ZEROS
            return c
        lax.fori_loop(0, nblk // _UNROLL, z, 0)

    def _treemax(vs):
        while len(vs) > 1:
            vs = [jnp.maximum(a, b) for a, b in zip(vs[::2], vs[1::2])]
        return vs[0]

    def pass_hist(row_ref, key_fn, sel_fn, record):
        def body(i, mx):
            ms = [monotone(row_ref[pl.ds((i * _UNROLL + u) * 16, 16)])
                  for u in range(_UNROLL)]
            cands = []
            for m in ms:
                kv = key_fn(m)
                if sel_fn is None:
                    plsc.addupdate_scatter(hist, [kv], ONES)
                    cands.append(kv)
                else:
                    s = sel_fn(m)
                    plsc.addupdate_scatter(hist, [kv], ONES, mask=s)
                    if record:
                        plsc.store_scatter(val, [kv], m, mask=s)
                    cands.append(jnp.where(s, kv, NEGINF))
            return jnp.maximum(mx, _treemax(cands))
        mx = lax.fori_loop(0, H // 16 // _UNROLL, body, NEGINF)
        return jnp.max(mx)

    def find_bucket(jstart, Kr):
        def cond(c):
            return jnp.logical_not(c[2])

        def bodyw(c):
            j, T, _ = c
            h = hist[pl.ds(j * 16, 16)]
            bs = jnp.sum(h)
            hit = (T + bs) >= Kr
            return (jnp.where(hit, j, j - 1), jnp.where(hit, T, T + bs), hit)

        j, T, _ = lax.while_loop(cond, bodyw,
                                 (jstart, jnp.int32(0), False))
        h = hist[pl.ds(j * 16, 16)]
        P = plsc.cumsum(h)
        total = jnp.max(P)
        cnt_ge = (T + total) - P + h
        maskv = cnt_ge >= Kr
        rev = lax.rev(maskv.astype(jnp.int32), (0,)) != 0
        l_vec = 15 - plsc.all_reduce_ffs(rev)
        Psel = jnp.max(jnp.where(lanes == l_vec, P, 0))
        hsel = jnp.max(jnp.where(lanes == l_vec, h, 0))
        cnt_gt = (T + total) - Psel
        return j * 16 + jnp.max(l_vec), Kr - cnt_gt, hsel

    def emit(m_thr, r_local):
        mv = m_thr + ZEROS
        yv = jnp.where(mv >= 0, mv, mv ^ 0x7FFFFFFF)
        plsc.store_scatter(stage, [r_local + ZEROS], yv, mask=lanes == 0)

    def process_row(row_ref, r_local):
        zero_hist(256)
        key1 = lambda m: lax.shift_right_arithmetic(m, 20) + 2048
        mx1 = pass_hist(row_ref, key1, None, False)
        b1, K1, _ = find_bucket(lax.shift_right_arithmetic(mx1, 4), K)

        zero_hist(256)
        key2 = lambda m: lax.shift_right_logical(m, 8) & 0xFFF
        mx2 = pass_hist(row_ref, key2, lambda m: key1(m) == b1, True)
        b2, K2, h2 = find_bucket(lax.shift_right_arithmetic(mx2, 4), K1)

        @pl.when(h2 == 1)
        def _():
            vv = val[pl.ds((b2 >> 4) << 4, 16)]
            emit(jnp.max(jnp.where(lanes == (b2 & 15), vv, NEGINF)), r_local)

        @pl.when(h2 > 1)
        def _():
            zero_hist(16)
            key3 = lambda m: m & 0xFF
            mx3 = pass_hist(row_ref, key3,
                            lambda m: (key1(m) == b1) & (key2(m) == b2),
                            False)
            b3, _, _ = find_bucket(lax.shift_right_arithmetic(mx3, 4), K2)
            emit(((b1 - 2048) << 20) | (b2 << 8) | b3, r_local)

    pltpu.make_async_copy(enc_hbm.at[base_row], rowa, sema).start()
    pltpu.make_async_copy(enc_hbm.at[base_row + 1], rowb, semb).start()

    def outer(i, c):
        for half, (buf, sem) in enumerate(((rowa, sema), (rowb, semb))):
            r = i * 2 + half
            pltpu.make_async_copy(enc_hbm.at[base_row + r], buf, sem).wait()
            process_row(buf, r)

            @pl.when(r + 2 < _RPW)
            def _():
                pltpu.make_async_copy(
                    enc_hbm.at[base_row + r + 2], buf, sem).start()
        return c

    lax.fori_loop(0, _RPW // 2, outer, 0)
    pltpu.sync_copy(stage, thr_hbm.at[pl.ds(base_row, _RPW)])


def _thresholds_sc(enc):
    mesh = plsc.VectorSubcoreMesh(core_axis_name="c", subcore_axis_name="s")
    k = functools.partial(
        pl.kernel,
        mesh=mesh,
        compiler_params=pltpu.CompilerParams(needs_layout_passes=False),
        out_type=jax.ShapeDtypeStruct((B,), jnp.int32),
        scratch_types=[
            pltpu.VMEM((H,), jnp.int32),
            pltpu.VMEM((H,), jnp.int32),
            pltpu.VMEM((4096,), jnp.int32),
            pltpu.VMEM((4096,), jnp.int32),
            pltpu.VMEM((_RPW,), jnp.int32),
            pltpu.SemaphoreType.DMA,
            pltpu.SemaphoreType.DMA,
        ],
    )(_sc_thr_body)
    enc_i32 = lax.bitcast_convert_type(enc, jnp.int32)
    return lax.bitcast_convert_type(k(enc_i32), jnp.float32)


_DEC_RBLK = 512
_DEC_HBLK = 512
_DEC_HSTEPS = H // _DEC_HBLK


def _dec_body(enc_ref, thr_ref, w2_ref, b2_ref, sp_ref, dec_ref):
    h = pl.program_id(1)
    sp = jnp.where(enc_ref[...] >= thr_ref[...], enc_ref[...], 0.0)
    sp_ref[...] = sp
    part = lax.dot_general(
        sp, w2_ref[...], (((1,), (1,)), ((), ())),
        preferred_element_type=jnp.float32,
    )

    @pl.when(h == 0)
    def _():
        dec_ref[...] = part

    @pl.when(h != 0)
    def _():
        dec_ref[...] += part

    @pl.when(h == _DEC_HSTEPS - 1)
    def _():
        dec_ref[...] = jax.nn.sigmoid(dec_ref[...] + b2_ref[...])


def _decoder(enc, thr, W2, b2_2d):
    grid = (B // _DEC_RBLK, _DEC_HSTEPS)
    return pl.pallas_call(
        _dec_body,
        grid=grid,
        in_specs=[
            pl.BlockSpec((_DEC_RBLK, _DEC_HBLK), lambda r, h: (r, h)),
            pl.BlockSpec((_DEC_RBLK, 1), lambda r, h: (r, 0)),
            pl.BlockSpec((D, _DEC_HBLK), lambda r, h: (0, h)),
            pl.BlockSpec((1, D), lambda r, h: (0, 0)),
        ],
        out_specs=[
            pl.BlockSpec((_DEC_RBLK, _DEC_HBLK), lambda r, h: (r, h)),
            pl.BlockSpec((_DEC_RBLK, D), lambda r, h: (r, 0)),
        ],
        out_shape=[
            jax.ShapeDtypeStruct((B, H), jnp.float32),
            jax.ShapeDtypeStruct((B, D), jnp.float32),
        ],
        compiler_params=pltpu.CompilerParams(
            dimension_semantics=("parallel", "arbitrary")),
    )(enc, thr, W2, b2_2d)


def kernel(x, W1, b1, W2, b2):
    x = x.reshape(B, D)
    enc = _encoder(x, W1, b1.reshape(1, H))
    thr = _thresholds_sc(enc).reshape(B, 1)
    sparse, decoded = _decoder(enc, thr, W2, b2.reshape(1, D))
    return decoded, sparse


# speedup vs baseline: 2.9978x; 1.2679x over previous
import functools

import jax
import jax.numpy as jnp
from jax import lax
from jax.experimental import pallas as pl
from jax.experimental.pallas import tpu as pltpu
from jax.experimental.pallas import tpu_sc as plsc

B = 4096
D = 2048
H = 8192
K = 256


_ENC_RBLK = 512
_ENC_HBLK = 1024


def _enc_body(x_ref, w_ref, b_ref, out_ref):
    acc = lax.dot_general(
        x_ref[...], w_ref[...], (((1,), (1,)), ((), ())),
        preferred_element_type=jnp.float32,
    )
    out_ref[...] = acc + b_ref[...]


def _encoder(x, W1, b1_2d):
    rows = x.shape[0]
    grid = (rows // _ENC_RBLK, H // _ENC_HBLK)
    return pl.pallas_call(
        _enc_body,
        grid=grid,
        in_specs=[
            pl.BlockSpec((_ENC_RBLK, D), lambda r, h: (r, 0)),
            pl.BlockSpec((_ENC_HBLK, D), lambda r, h: (h, 0)),
            pl.BlockSpec((1, _ENC_HBLK), lambda r, h: (0, h)),
        ],
        out_specs=pl.BlockSpec((_ENC_RBLK, _ENC_HBLK), lambda r, h: (r, h)),
        out_shape=jax.ShapeDtypeStruct((rows, H), jnp.float32),
        compiler_params=pltpu.CompilerParams(
            dimension_semantics=("parallel", "arbitrary")),
    )(x, W1, b1_2d)


_THR_RBLK = 256


def _thr_body(enc_ref, thr_ref, mu_ref):
    y = lax.bitcast_convert_type(enc_ref[...], jnp.uint32)
    neg = y >= jnp.uint32(0x80000000)
    mu = jnp.where(neg, ~y, y | jnp.uint32(0x80000000))
    mu_ref[...] = mu

    lo0 = jnp.zeros((_THR_RBLK, 1), jnp.uint32)
    hi0 = jnp.full((_THR_RBLK, 1), jnp.uint32(0xFFFFFFFF))

    def step(_, carry):
        lo, hi = carry
        mid = lo + ((hi - lo) // jnp.uint32(2)) + ((hi - lo) % jnp.uint32(2))
        cnt = jnp.sum((mu_ref[...] >= mid).astype(jnp.int32), axis=1,
                      keepdims=True)
        ge = cnt >= K
        lo = jnp.where(ge, mid, lo)
        hi = jnp.where(ge, hi, mid - jnp.uint32(1))
        return lo, hi

    lo, hi = lax.fori_loop(0, 32, step, (lo0, hi0))
    code = lo
    pos = code >= jnp.uint32(0x80000000)
    y_out = jnp.where(pos, code ^ jnp.uint32(0x80000000), ~code)
    thr_ref[...] = lax.bitcast_convert_type(y_out, jnp.float32)


def _thresholds(enc):
    return pl.pallas_call(
        _thr_body,
        grid=(B // _THR_RBLK,),
        in_specs=[pl.BlockSpec((_THR_RBLK, H), lambda r: (r, 0))],
        out_specs=pl.BlockSpec((_THR_RBLK, 1), lambda r: (r, 0)),
        out_shape=jax.ShapeDtypeStruct((B, 1), jnp.float32),
        scratch_shapes=[pltpu.VMEM((_THR_RBLK, H), jnp.uint32)],
        compiler_params=pltpu.CompilerParams(
            dimension_semantics=("arbitrary",)),
    )(enc)


_NW = 32
_RPW = B // _NW


_UNROLL = 8


def _sc_thr_body(rpw, enc_hbm, thr_hbm, rowa, rowb, hist, val, stage, sema,
                 semb):
    cid = lax.axis_index("c")
    sid = lax.axis_index("s")
    wid = sid * 2 + cid
    base_row = wid * rpw

    lanes = lax.iota(jnp.int32, 16)
    ONES = jnp.ones((16,), jnp.int32)
    ZEROS = jnp.zeros((16,), jnp.int32)
    NEGINF = jnp.full((16,), jnp.int32(-2147483648))

    def monotone(y):
        return y ^ lax.shift_right_logical(
            lax.shift_right_arithmetic(y, 31), 1)

    def zero_hist(nblk):
        def z(i, c):
            for u in range(_UNROLL):
                hist[pl.ds((i * _UNROLL + u) * 16, 16)] = ZEROS
            return c
        lax.fori_loop(0, nblk // _UNROLL, z, 0)

    def _treemax(vs):
        while len(vs) > 1:
            vs = [jnp.maximum(a, b) for a, b in zip(vs[::2], vs[1::2])]
        return vs[0]

    def pass_hist(row_ref, key_fn, sel_fn, record):
        def body(i, mx):
            ms = [monotone(row_ref[pl.ds((i * _UNROLL + u) * 16, 16)])
                  for u in range(_UNROLL)]
            cands = []
            for m in ms:
                kv = key_fn(m)
                if sel_fn is None:
                    plsc.addupdate_scatter(hist, [kv], ONES)
                    cands.append(kv)
                else:
                    s = sel_fn(m)
                    plsc.addupdate_scatter(hist, [kv], ONES, mask=s)
                    if record:
                        plsc.store_scatter(val, [kv], m, mask=s)
                    cands.append(jnp.where(s, kv, NEGINF))
            return jnp.maximum(mx, _treemax(cands))
        mx = lax.fori_loop(0, H // 16 // _UNROLL, body, NEGINF)
        return jnp.max(mx)

    def find_bucket(jstart, Kr):
        def cond(c):
            return jnp.logical_not(c[2])

        def bodyw(c):
            j, T, _ = c
            h = hist[pl.ds(j * 16, 16)]
            bs = jnp.sum(h)
            hit = (T + bs) >= Kr
            return (jnp.where(hit, j, j - 1), jnp.where(hit, T, T + bs), hit)

        j, T, _ = lax.while_loop(cond, bodyw,
                                 (jstart, jnp.int32(0), False))
        h = hist[pl.ds(j * 16, 16)]
        P = plsc.cumsum(h)
        total = jnp.max(P)
        cnt_ge = (T + total) - P + h
        maskv = cnt_ge >= Kr
        rev = lax.rev(maskv.astype(jnp.int32), (0,)) != 0
        l_vec = 15 - plsc.all_reduce_ffs(rev)
        Psel = jnp.max(jnp.where(lanes == l_vec, P, 0))
        hsel = jnp.max(jnp.where(lanes == l_vec, h, 0))
        cnt_gt = (T + total) - Psel
        return j * 16 + jnp.max(l_vec), Kr - cnt_gt, hsel

    def emit(m_thr, r_local):
        mv = m_thr + ZEROS
        yv = jnp.where(mv >= 0, mv, mv ^ 0x7FFFFFFF)
        plsc.store_scatter(stage, [r_local + ZEROS], yv, mask=lanes == 0)

    def process_row(row_ref, r_local):
        zero_hist(256)
        key1 = lambda m: lax.shift_right_arithmetic(m, 20) + 2048
        mx1 = pass_hist(row_ref, key1, None, False)
        b1, K1, _ = find_bucket(lax.shift_right_arithmetic(mx1, 4), K)

        zero_hist(256)
        key2 = lambda m: lax.shift_right_logical(m, 8) & 0xFFF
        mx2 = pass_hist(row_ref, key2, lambda m: key1(m) == b1, True)
        b2, K2, h2 = find_bucket(lax.shift_right_arithmetic(mx2, 4), K1)

        @pl.when(h2 == 1)
        def _():
            vv = val[pl.ds((b2 >> 4) << 4, 16)]
            emit(jnp.max(jnp.where(lanes == (b2 & 15), vv, NEGINF)), r_local)

        @pl.when(h2 > 1)
        def _():
            zero_hist(16)
            key3 = lambda m: m & 0xFF
            mx3 = pass_hist(row_ref, key3,
                            lambda m: (key1(m) == b1) & (key2(m) == b2),
                            False)
            b3, _, _ = find_bucket(lax.shift_right_arithmetic(mx3, 4), K2)
            emit(((b1 - 2048) << 20) | (b2 << 8) | b3, r_local)

    pltpu.make_async_copy(enc_hbm.at[base_row], rowa, sema).start()
    pltpu.make_async_copy(enc_hbm.at[base_row + 1], rowb, semb).start()

    def outer(i, c):
        for half, (buf, sem) in enumerate(((rowa, sema), (rowb, semb))):
            r = i * 2 + half
            pltpu.make_async_copy(enc_hbm.at[base_row + r], buf, sem).wait()
            process_row(buf, r)

            @pl.when(r + 2 < rpw)
            def _():
                pltpu.make_async_copy(
                    enc_hbm.at[base_row + r + 2], buf, sem).start()
        return c

    lax.fori_loop(0, rpw // 2, outer, 0)
    pltpu.sync_copy(stage, thr_hbm.at[pl.ds(base_row, rpw)])


def _thresholds_sc(enc):
    rows = enc.shape[0]
    rpw = rows // _NW
    mesh = plsc.VectorSubcoreMesh(core_axis_name="c", subcore_axis_name="s")
    k = functools.partial(
        pl.kernel,
        mesh=mesh,
        compiler_params=pltpu.CompilerParams(needs_layout_passes=False),
        out_type=jax.ShapeDtypeStruct((rows,), jnp.int32),
        scratch_types=[
            pltpu.VMEM((H,), jnp.int32),
            pltpu.VMEM((H,), jnp.int32),
            pltpu.VMEM((4096,), jnp.int32),
            pltpu.VMEM((4096,), jnp.int32),
            pltpu.VMEM((rpw,), jnp.int32),
            pltpu.SemaphoreType.DMA,
            pltpu.SemaphoreType.DMA,
        ],
    )(functools.partial(_sc_thr_body, rpw))
    enc_i32 = lax.bitcast_convert_type(enc, jnp.int32)
    return lax.bitcast_convert_type(k(enc_i32), jnp.float32)


_DEC_RBLK = 512
_DEC_HBLK = 512
_DEC_HSTEPS = H // _DEC_HBLK


def _dec_body(enc_ref, thr_ref, w2_ref, b2_ref, sp_ref, dec_ref):
    h = pl.program_id(1)
    sp = jnp.where(enc_ref[...] >= thr_ref[...], enc_ref[...], 0.0)
    sp_ref[...] = sp
    part = lax.dot_general(
        sp, w2_ref[...], (((1,), (1,)), ((), ())),
        preferred_element_type=jnp.float32,
    )

    @pl.when(h == 0)
    def _():
        dec_ref[...] = part

    @pl.when(h != 0)
    def _():
        dec_ref[...] += part

    @pl.when(h == _DEC_HSTEPS - 1)
    def _():
        dec_ref[...] = jax.nn.sigmoid(dec_ref[...] + b2_ref[...])


def _decoder(enc, thr, W2, b2_2d):
    rows = enc.shape[0]
    grid = (rows // _DEC_RBLK, _DEC_HSTEPS)
    return pl.pallas_call(
        _dec_body,
        grid=grid,
        in_specs=[
            pl.BlockSpec((_DEC_RBLK, _DEC_HBLK), lambda r, h: (r, h)),
            pl.BlockSpec((_DEC_RBLK, 1), lambda r, h: (r, 0)),
            pl.BlockSpec((D, _DEC_HBLK), lambda r, h: (0, h)),
            pl.BlockSpec((1, D), lambda r, h: (0, 0)),
        ],
        out_specs=[
            pl.BlockSpec((_DEC_RBLK, _DEC_HBLK), lambda r, h: (r, h)),
            pl.BlockSpec((_DEC_RBLK, D), lambda r, h: (r, 0)),
        ],
        out_shape=[
            jax.ShapeDtypeStruct((rows, H), jnp.float32),
            jax.ShapeDtypeStruct((rows, D), jnp.float32),
        ],
        compiler_params=pltpu.CompilerParams(
            dimension_semantics=("parallel", "arbitrary")),
    )(enc, thr, W2, b2_2d)


_NCHUNK = 4
_CB = B // _NCHUNK


def kernel(x, W1, b1, W2, b2):
    x = x.reshape(B, D)
    b1_2d = b1.reshape(1, H)
    b2_2d = b2.reshape(1, D)
    encs = [_encoder(x[c * _CB:(c + 1) * _CB], W1, b1_2d)
            for c in range(_NCHUNK)]
    thrs = [_thresholds_sc(e).reshape(_CB, 1) for e in encs]
    outs = [_decoder(e, t, W2, b2_2d) for e, t in zip(encs, thrs)]
    sparse = jnp.concatenate([o[0] for o in outs], axis=0)
    decoded = jnp.concatenate([o[1] for o in outs], axis=0)
    return decoded, sparse


# speedup vs baseline: 3.0050x; 1.0024x over previous
import functools

import jax
import jax.numpy as jnp
from jax import lax
from jax.experimental import pallas as pl
from jax.experimental.pallas import tpu as pltpu
from jax.experimental.pallas import tpu_sc as plsc

B = 4096
D = 2048
H = 8192
K = 256


_ENC_RBLK = 512
_ENC_HBLK = 1024


def _enc_body(x_ref, w_ref, b_ref, out_ref):
    acc = lax.dot_general(
        x_ref[...], w_ref[...], (((1,), (1,)), ((), ())),
        preferred_element_type=jnp.float32,
    )
    out_ref[...] = acc + b_ref[...]


def _encoder(x, W1, b1_2d):
    rows = x.shape[0]
    grid = (rows // _ENC_RBLK, H // _ENC_HBLK)
    return pl.pallas_call(
        _enc_body,
        grid=grid,
        in_specs=[
            pl.BlockSpec((_ENC_RBLK, D), lambda r, h: (r, 0)),
            pl.BlockSpec((_ENC_HBLK, D), lambda r, h: (h, 0)),
            pl.BlockSpec((1, _ENC_HBLK), lambda r, h: (0, h)),
        ],
        out_specs=pl.BlockSpec((_ENC_RBLK, _ENC_HBLK), lambda r, h: (r, h)),
        out_shape=jax.ShapeDtypeStruct((rows, H), jnp.float32),
        compiler_params=pltpu.CompilerParams(
            dimension_semantics=("parallel", "arbitrary")),
    )(x, W1, b1_2d)


_THR_RBLK = 256


def _thr_body(enc_ref, thr_ref, mu_ref):
    y = lax.bitcast_convert_type(enc_ref[...], jnp.uint32)
    neg = y >= jnp.uint32(0x80000000)
    mu = jnp.where(neg, ~y, y | jnp.uint32(0x80000000))
    mu_ref[...] = mu

    lo0 = jnp.zeros((_THR_RBLK, 1), jnp.uint32)
    hi0 = jnp.full((_THR_RBLK, 1), jnp.uint32(0xFFFFFFFF))

    def step(_, carry):
        lo, hi = carry
        mid = lo + ((hi - lo) // jnp.uint32(2)) + ((hi - lo) % jnp.uint32(2))
        cnt = jnp.sum((mu_ref[...] >= mid).astype(jnp.int32), axis=1,
                      keepdims=True)
        ge = cnt >= K
        lo = jnp.where(ge, mid, lo)
        hi = jnp.where(ge, hi, mid - jnp.uint32(1))
        return lo, hi

    lo, hi = lax.fori_loop(0, 32, step, (lo0, hi0))
    code = lo
    pos = code >= jnp.uint32(0x80000000)
    y_out = jnp.where(pos, code ^ jnp.uint32(0x80000000), ~code)
    thr_ref[...] = lax.bitcast_convert_type(y_out, jnp.float32)


def _thresholds(enc):
    return pl.pallas_call(
        _thr_body,
        grid=(B // _THR_RBLK,),
        in_specs=[pl.BlockSpec((_THR_RBLK, H), lambda r: (r, 0))],
        out_specs=pl.BlockSpec((_THR_RBLK, 1), lambda r: (r, 0)),
        out_shape=jax.ShapeDtypeStruct((B, 1), jnp.float32),
        scratch_shapes=[pltpu.VMEM((_THR_RBLK, H), jnp.uint32)],
        compiler_params=pltpu.CompilerParams(
            dimension_semantics=("arbitrary",)),
    )(enc)


_NW = 32
_RPW = B // _NW


_UNROLL = 8


def _sc_thr_body(rpw, enc_hbm, thr_hbm, rowa, rowb, hist, val, stage, sema,
                 semb):
    cid = lax.axis_index("c")
    sid = lax.axis_index("s")
    wid = sid * 2 + cid
    base_row = wid * rpw

    lanes = lax.iota(jnp.int32, 16)
    ONES = jnp.ones((16,), jnp.int32)
    ZEROS = jnp.zeros((16,), jnp.int32)
    NEGINF = jnp.full((16,), jnp.int32(-2147483648))

    def monotone(y):
        return y ^ lax.shift_right_logical(
            lax.shift_right_arithmetic(y, 31), 1)

    def zero_hist(nblk):
        def z(i, c):
            for u in range(_UNROLL):
                hist[pl.ds((i * _UNROLL + u) * 16, 16)] = ZEROS
            return c
        lax.fori_loop(0, nblk // _UNROLL, z, 0)

    def _treemax(vs):
        while len(vs) > 1:
            vs = [jnp.maximum(a, b) for a, b in zip(vs[::2], vs[1::2])]
        return vs[0]

    LSPREAD = (lax.iota(jnp.int32, 16) & 1) << 12

    def pass_hist(row_ref, key_fn, sel_fn, record):
        def body(i, mx):
            ms = [monotone(row_ref[pl.ds((i * _UNROLL + u) * 16, 16)])
                  for u in range(_UNROLL)]
            cands = []
            for m in ms:
                kv = key_fn(m)
                if sel_fn is None:
                    plsc.addupdate_scatter(hist, [kv + LSPREAD], ONES)
                    cands.append(kv)
                else:
                    s = sel_fn(m)
                    plsc.addupdate_scatter(hist, [kv], ONES, mask=s)
                    if record:
                        plsc.store_scatter(val, [kv], m, mask=s)
                    cands.append(jnp.where(s, kv, NEGINF))
            return jnp.maximum(mx, _treemax(cands))
        mx = lax.fori_loop(0, H // 16 // _UNROLL, body, NEGINF)
        return jnp.max(mx)

    def find_bucket(jstart, Kr, spread=False):
        def hblk(j):
            h = hist[pl.ds(j * 16, 16)]
            if spread:
                h = h + hist[pl.ds(4096 + j * 16, 16)]
            return h

        def cond(c):
            return jnp.logical_not(c[2])

        def bodyw(c):
            j, T, _ = c
            bs = jnp.sum(hblk(j))
            hit = (T + bs) >= Kr
            return (jnp.where(hit, j, j - 1), jnp.where(hit, T, T + bs), hit)

        j, T, _ = lax.while_loop(cond, bodyw,
                                 (jstart, jnp.int32(0), False))
        h = hblk(j)
        P = plsc.cumsum(h)
        total = jnp.max(P)
        cnt_ge = (T + total) - P + h
        maskv = cnt_ge >= Kr
        rev = lax.rev(maskv.astype(jnp.int32), (0,)) != 0
        l_vec = 15 - plsc.all_reduce_ffs(rev)
        Psel = jnp.max(jnp.where(lanes == l_vec, P, 0))
        hsel = jnp.max(jnp.where(lanes == l_vec, h, 0))
        cnt_gt = (T + total) - Psel
        return j * 16 + jnp.max(l_vec), Kr - cnt_gt, hsel

    def emit(m_thr, r_local):
        mv = m_thr + ZEROS
        yv = jnp.where(mv >= 0, mv, mv ^ 0x7FFFFFFF)
        plsc.store_scatter(stage, [r_local + ZEROS], yv, mask=lanes == 0)

    def process_row(row_ref, r_local):
        zero_hist(512)
        key1 = lambda m: lax.shift_right_arithmetic(m, 20) + 2048
        mx1 = pass_hist(row_ref, key1, None, False)
        b1, K1, _ = find_bucket(lax.shift_right_arithmetic(mx1, 4), K,
                                spread=True)

        zero_hist(256)
        key2 = lambda m: lax.shift_right_logical(m, 8) & 0xFFF
        mx2 = pass_hist(row_ref, key2, lambda m: key1(m) == b1, True)
        b2, K2, h2 = find_bucket(lax.shift_right_arithmetic(mx2, 4), K1)

        @pl.when(h2 == 1)
        def _():
            vv = val[pl.ds((b2 >> 4) << 4, 16)]
            emit(jnp.max(jnp.where(lanes == (b2 & 15), vv, NEGINF)), r_local)

        @pl.when(h2 > 1)
        def _():
            zero_hist(16)
            key3 = lambda m: m & 0xFF
            mx3 = pass_hist(row_ref, key3,
                            lambda m: (key1(m) == b1) & (key2(m) == b2),
                            False)
            b3, _, _ = find_bucket(lax.shift_right_arithmetic(mx3, 4), K2)
            emit(((b1 - 2048) << 20) | (b2 << 8) | b3, r_local)

    pltpu.make_async_copy(enc_hbm.at[base_row], rowa, sema).start()
    pltpu.make_async_copy(enc_hbm.at[base_row + 1], rowb, semb).start()

    def outer(i, c):
        for half, (buf, sem) in enumerate(((rowa, sema), (rowb, semb))):
            r = i * 2 + half
            pltpu.make_async_copy(enc_hbm.at[base_row + r], buf, sem).wait()
            process_row(buf, r)

            @pl.when(r + 2 < rpw)
            def _():
                pltpu.make_async_copy(
                    enc_hbm.at[base_row + r + 2], buf, sem).start()
        return c

    lax.fori_loop(0, rpw // 2, outer, 0)
    pltpu.sync_copy(stage, thr_hbm.at[pl.ds(base_row, rpw)])


def _thresholds_sc(enc):
    rows = enc.shape[0]
    rpw = rows // _NW
    mesh = plsc.VectorSubcoreMesh(core_axis_name="c", subcore_axis_name="s")
    k = functools.partial(
        pl.kernel,
        mesh=mesh,
        compiler_params=pltpu.CompilerParams(needs_layout_passes=False),
        out_type=jax.ShapeDtypeStruct((rows,), jnp.int32),
        scratch_types=[
            pltpu.VMEM((H,), jnp.int32),
            pltpu.VMEM((H,), jnp.int32),
            pltpu.VMEM((8192,), jnp.int32),
            pltpu.VMEM((4096,), jnp.int32),
            pltpu.VMEM((rpw,), jnp.int32),
            pltpu.SemaphoreType.DMA,
            pltpu.SemaphoreType.DMA,
        ],
    )(functools.partial(_sc_thr_body, rpw))
    enc_i32 = lax.bitcast_convert_type(enc, jnp.int32)
    return lax.bitcast_convert_type(k(enc_i32), jnp.float32)


_DEC_RBLK = 512
_DEC_HBLK = 512
_DEC_HSTEPS = H // _DEC_HBLK


def _dec_body(enc_ref, thr_ref, w2_ref, b2_ref, sp_ref, dec_ref):
    h = pl.program_id(1)
    sp = jnp.where(enc_ref[...] >= thr_ref[...], enc_ref[...], 0.0)
    sp_ref[...] = sp
    part = lax.dot_general(
        sp, w2_ref[...], (((1,), (1,)), ((), ())),
        preferred_element_type=jnp.float32,
    )

    @pl.when(h == 0)
    def _():
        dec_ref[...] = part

    @pl.when(h != 0)
    def _():
        dec_ref[...] += part

    @pl.when(h == _DEC_HSTEPS - 1)
    def _():
        dec_ref[...] = jax.nn.sigmoid(dec_ref[...] + b2_ref[...])


def _decoder(enc, thr, W2, b2_2d):
    rows = enc.shape[0]
    grid = (rows // _DEC_RBLK, _DEC_HSTEPS)
    return pl.pallas_call(
        _dec_body,
        grid=grid,
        in_specs=[
            pl.BlockSpec((_DEC_RBLK, _DEC_HBLK), lambda r, h: (r, h)),
            pl.BlockSpec((_DEC_RBLK, 1), lambda r, h: (r, 0)),
            pl.BlockSpec((D, _DEC_HBLK), lambda r, h: (0, h)),
            pl.BlockSpec((1, D), lambda r, h: (0, 0)),
        ],
        out_specs=[
            pl.BlockSpec((_DEC_RBLK, _DEC_HBLK), lambda r, h: (r, h)),
            pl.BlockSpec((_DEC_RBLK, D), lambda r, h: (r, 0)),
        ],
        out_shape=[
            jax.ShapeDtypeStruct((rows, H), jnp.float32),
            jax.ShapeDtypeStruct((rows, D), jnp.float32),
        ],
        compiler_params=pltpu.CompilerParams(
            dimension_semantics=("parallel", "arbitrary")),
    )(enc, thr, W2, b2_2d)


_NCHUNK = 8
_CB = B // _NCHUNK


def kernel(x, W1, b1, W2, b2):
    x = x.reshape(B, D)
    b1_2d = b1.reshape(1, H)
    b2_2d = b2.reshape(1, D)
    encs = [_encoder(x[c * _CB:(c + 1) * _CB], W1, b1_2d)
            for c in range(_NCHUNK)]
    thrs = [_thresholds_sc(e).reshape(_CB, 1) for e in encs]
    outs = [_decoder(e, t, W2, b2_2d) for e, t in zip(encs, thrs)]
    sparse = jnp.concatenate([o[0] for o in outs], axis=0)
    decoded = jnp.concatenate([o[1] for o in outs], axis=0)
    return decoded, sparse


# speedup vs baseline: 3.2389x; 1.0778x over previous
import functools

import jax
import jax.numpy as jnp
from jax import lax
from jax.experimental import pallas as pl
from jax.experimental.pallas import tpu as pltpu
from jax.experimental.pallas import tpu_sc as plsc

B = 4096
D = 2048
H = 8192
K = 256


_ENC_RBLK = 512
_ENC_HBLK = 1024


def _enc_body(x_ref, w_ref, b_ref, out_ref):
    acc = lax.dot_general(
        x_ref[...], w_ref[...], (((1,), (1,)), ((), ())),
        preferred_element_type=jnp.float32,
    )
    out_ref[...] = acc + b_ref[...]


def _encoder(x, W1, b1_2d):
    rows = x.shape[0]
    grid = (rows // _ENC_RBLK, H // _ENC_HBLK)
    return pl.pallas_call(
        _enc_body,
        grid=grid,
        in_specs=[
            pl.BlockSpec((_ENC_RBLK, D), lambda r, h: (r, 0)),
            pl.BlockSpec((_ENC_HBLK, D), lambda r, h: (h, 0)),
            pl.BlockSpec((1, _ENC_HBLK), lambda r, h: (0, h)),
        ],
        out_specs=pl.BlockSpec((_ENC_RBLK, _ENC_HBLK), lambda r, h: (r, h)),
        out_shape=jax.ShapeDtypeStruct((rows, H), jnp.float32),
        compiler_params=pltpu.CompilerParams(
            dimension_semantics=("parallel", "arbitrary")),
    )(x, W1, b1_2d)


_THR_RBLK = 256


def _thr_body(enc_ref, thr_ref, mu_ref):
    y = lax.bitcast_convert_type(enc_ref[...], jnp.uint32)
    neg = y >= jnp.uint32(0x80000000)
    mu = jnp.where(neg, ~y, y | jnp.uint32(0x80000000))
    mu_ref[...] = mu

    lo0 = jnp.zeros((_THR_RBLK, 1), jnp.uint32)
    hi0 = jnp.full((_THR_RBLK, 1), jnp.uint32(0xFFFFFFFF))

    def step(_, carry):
        lo, hi = carry
        mid = lo + ((hi - lo) // jnp.uint32(2)) + ((hi - lo) % jnp.uint32(2))
        cnt = jnp.sum((mu_ref[...] >= mid).astype(jnp.int32), axis=1,
                      keepdims=True)
        ge = cnt >= K
        lo = jnp.where(ge, mid, lo)
        hi = jnp.where(ge, hi, mid - jnp.uint32(1))
        return lo, hi

    lo, hi = lax.fori_loop(0, 32, step, (lo0, hi0))
    code = lo
    pos = code >= jnp.uint32(0x80000000)
    y_out = jnp.where(pos, code ^ jnp.uint32(0x80000000), ~code)
    thr_ref[...] = lax.bitcast_convert_type(y_out, jnp.float32)


def _thresholds(enc):
    return pl.pallas_call(
        _thr_body,
        grid=(B // _THR_RBLK,),
        in_specs=[pl.BlockSpec((_THR_RBLK, H), lambda r: (r, 0))],
        out_specs=pl.BlockSpec((_THR_RBLK, 1), lambda r: (r, 0)),
        out_shape=jax.ShapeDtypeStruct((B, 1), jnp.float32),
        scratch_shapes=[pltpu.VMEM((_THR_RBLK, H), jnp.uint32)],
        compiler_params=pltpu.CompilerParams(
            dimension_semantics=("arbitrary",)),
    )(enc)


_NW = 32
_RPW = B // _NW


_UNROLL = 8


def _sc_thr_body(rpw, enc_hbm, thr_hbm, rowa, rowb, hist, val, stage, sema,
                 semb):
    cid = lax.axis_index("c")
    sid = lax.axis_index("s")
    wid = sid * 2 + cid
    base_row = wid * rpw

    lanes = lax.iota(jnp.int32, 16)
    ONES = jnp.ones((16,), jnp.int32)
    ZEROS = jnp.zeros((16,), jnp.int32)
    NEGINF = jnp.full((16,), jnp.int32(-2147483648))

    def monotone(y):
        return y ^ lax.shift_right_logical(
            lax.shift_right_arithmetic(y, 31), 1)

    def zero_hist(nblk):
        @plsc.parallel_loop(0, nblk, unroll=_UNROLL)
        def _(i):
            hist[pl.ds(i * 16, 16)] = ZEROS

    def _treemax(vs):
        while len(vs) > 1:
            vs = [jnp.maximum(a, b) for a, b in zip(vs[::2], vs[1::2])]
        return vs[0]

    LSPREAD = (lax.iota(jnp.int32, 16) & 1) << 12

    def pass_hist(row_ref, key_fn, sel_fn, record):
        @plsc.parallel_loop(0, H // 16, unroll=_UNROLL, carry=NEGINF)
        def mx(i, mxc):
            m = monotone(row_ref[pl.ds(i * 16, 16)])
            kv = key_fn(m)
            if sel_fn is None:
                plsc.addupdate_scatter(hist, [kv + LSPREAD], ONES)
                return jnp.maximum(mxc, kv)
            s = sel_fn(m)
            plsc.addupdate_scatter(hist, [kv], ONES, mask=s)
            if record:
                plsc.store_scatter(val, [kv], m, mask=s)
            return jnp.maximum(mxc, jnp.where(s, kv, NEGINF))
        return jnp.max(mx)

    def find_bucket(jstart, Kr, spread=False):
        def hblk(j):
            h = hist[pl.ds(j * 16, 16)]
            if spread:
                h = h + hist[pl.ds(4096 + j * 16, 16)]
            return h

        def cond(c):
            return jnp.logical_not(c[2])

        def bodyw(c):
            j, T, _ = c
            bs = jnp.sum(hblk(j))
            hit = (T + bs) >= Kr
            return (jnp.where(hit, j, j - 1), jnp.where(hit, T, T + bs), hit)

        j, T, _ = lax.while_loop(cond, bodyw,
                                 (jstart, jnp.int32(0), False))
        h = hblk(j)
        P = plsc.cumsum(h)
        total = jnp.max(P)
        cnt_ge = (T + total) - P + h
        maskv = cnt_ge >= Kr
        rev = lax.rev(maskv.astype(jnp.int32), (0,)) != 0
        l_vec = 15 - plsc.all_reduce_ffs(rev)
        Psel = jnp.max(jnp.where(lanes == l_vec, P, 0))
        hsel = jnp.max(jnp.where(lanes == l_vec, h, 0))
        cnt_gt = (T + total) - Psel
        return j * 16 + jnp.max(l_vec), Kr - cnt_gt, hsel

    def emit(m_thr, r_local):
        mv = m_thr + ZEROS
        yv = jnp.where(mv >= 0, mv, mv ^ 0x7FFFFFFF)
        plsc.store_scatter(stage, [r_local + ZEROS], yv, mask=lanes == 0)

    def process_row(row_ref, r_local):
        zero_hist(512)
        key1 = lambda m: lax.shift_right_arithmetic(m, 20) + 2048
        mx1 = pass_hist(row_ref, key1, None, False)
        b1, K1, _ = find_bucket(lax.shift_right_arithmetic(mx1, 4), K,
                                spread=True)

        zero_hist(256)
        key2 = lambda m: lax.shift_right_logical(m, 8) & 0xFFF
        mx2 = pass_hist(row_ref, key2, lambda m: key1(m) == b1, True)
        b2, K2, h2 = find_bucket(lax.shift_right_arithmetic(mx2, 4), K1)

        @pl.when(h2 == 1)
        def _():
            vv = val[pl.ds((b2 >> 4) << 4, 16)]
            emit(jnp.max(jnp.where(lanes == (b2 & 15), vv, NEGINF)), r_local)

        @pl.when(h2 > 1)
        def _():
            zero_hist(16)
            key3 = lambda m: m & 0xFF
            mx3 = pass_hist(row_ref, key3,
                            lambda m: (key1(m) == b1) & (key2(m) == b2),
                            False)
            b3, _, _ = find_bucket(lax.shift_right_arithmetic(mx3, 4), K2)
            emit(((b1 - 2048) << 20) | (b2 << 8) | b3, r_local)

    pltpu.make_async_copy(enc_hbm.at[base_row], rowa, sema).start()
    pltpu.make_async_copy(enc_hbm.at[base_row + 1], rowb, semb).start()

    def outer(i, c):
        for half, (buf, sem) in enumerate(((rowa, sema), (rowb, semb))):
            r = i * 2 + half
            pltpu.make_async_copy(enc_hbm.at[base_row + r], buf, sem).wait()
            process_row(buf, r)

            @pl.when(r + 2 < rpw)
            def _():
                pltpu.make_async_copy(
                    enc_hbm.at[base_row + r + 2], buf, sem).start()
        return c

    lax.fori_loop(0, rpw // 2, outer, 0)
    pltpu.sync_copy(stage, thr_hbm.at[pl.ds(base_row, rpw)])


def _thresholds_sc(enc):
    rows = enc.shape[0]
    rpw = rows // _NW
    mesh = plsc.VectorSubcoreMesh(core_axis_name="c", subcore_axis_name="s")
    k = functools.partial(
        pl.kernel,
        mesh=mesh,
        compiler_params=pltpu.CompilerParams(needs_layout_passes=False),
        out_type=jax.ShapeDtypeStruct((rows,), jnp.int32),
        scratch_types=[
            pltpu.VMEM((H,), jnp.int32),
            pltpu.VMEM((H,), jnp.int32),
            pltpu.VMEM((8192,), jnp.int32),
            pltpu.VMEM((4096,), jnp.int32),
            pltpu.VMEM((rpw,), jnp.int32),
            pltpu.SemaphoreType.DMA,
            pltpu.SemaphoreType.DMA,
        ],
    )(functools.partial(_sc_thr_body, rpw))
    enc_i32 = lax.bitcast_convert_type(enc, jnp.int32)
    return lax.bitcast_convert_type(k(enc_i32), jnp.float32)


_DEC_RBLK = 512
_DEC_HBLK = 512
_DEC_HSTEPS = H // _DEC_HBLK


def _dec_body(enc_ref, thr_ref, w2_ref, b2_ref, sp_ref, dec_ref):
    h = pl.program_id(1)
    sp = jnp.where(enc_ref[...] >= thr_ref[...], enc_ref[...], 0.0)
    sp_ref[...] = sp
    part = lax.dot_general(
        sp, w2_ref[...], (((1,), (1,)), ((), ())),
        preferred_element_type=jnp.float32,
    )

    @pl.when(h == 0)
    def _():
        dec_ref[...] = part

    @pl.when(h != 0)
    def _():
        dec_ref[...] += part

    @pl.when(h == _DEC_HSTEPS - 1)
    def _():
        dec_ref[...] = jax.nn.sigmoid(dec_ref[...] + b2_ref[...])


def _decoder(enc, thr, W2, b2_2d):
    rows = enc.shape[0]
    grid = (rows // _DEC_RBLK, _DEC_HSTEPS)
    return pl.pallas_call(
        _dec_body,
        grid=grid,
        in_specs=[
            pl.BlockSpec((_DEC_RBLK, _DEC_HBLK), lambda r, h: (r, h)),
            pl.BlockSpec((_DEC_RBLK, 1), lambda r, h: (r, 0)),
            pl.BlockSpec((D, _DEC_HBLK), lambda r, h: (0, h)),
            pl.BlockSpec((1, D), lambda r, h: (0, 0)),
        ],
        out_specs=[
            pl.BlockSpec((_DEC_RBLK, _DEC_HBLK), lambda r, h: (r, h)),
            pl.BlockSpec((_DEC_RBLK, D), lambda r, h: (r, 0)),
        ],
        out_shape=[
            jax.ShapeDtypeStruct((rows, H), jnp.float32),
            jax.ShapeDtypeStruct((rows, D), jnp.float32),
        ],
        compiler_params=pltpu.CompilerParams(
            dimension_semantics=("parallel", "arbitrary")),
    )(enc, thr, W2, b2_2d)


_NCHUNK = 8
_CB = B // _NCHUNK


def kernel(x, W1, b1, W2, b2):
    x = x.reshape(B, D)
    b1_2d = b1.reshape(1, H)
    b2_2d = b2.reshape(1, D)
    encs = [_encoder(x[c * _CB:(c + 1) * _CB], W1, b1_2d)
            for c in range(_NCHUNK)]
    thrs = [_thresholds_sc(e).reshape(_CB, 1) for e in encs]
    outs = [_decoder(e, t, W2, b2_2d) for e, t in zip(encs, thrs)]
    sparse = jnp.concatenate([o[0] for o in outs], axis=0)
    decoded = jnp.concatenate([o[1] for o in outs], axis=0)
    return decoded, sparse
